# baseline (speedup 1.0000x reference)
import functools

import jax
import jax.numpy as jnp
from jax import lax
from jax.experimental import pallas as pl
from jax.experimental.pallas import tpu as pltpu
from jax.experimental.pallas import tpu_sc as plsc

_C = 10
_F = 128
_NC = 2
_NS = 16
_NW = _NC * _NS
_L = 16


def _sqrt16(v):
    bits = lax.bitcast_convert_type(v, jnp.int32)
    y = lax.bitcast_convert_type(
        (bits >> 1) + jnp.int32(0x1FBD1DF5), jnp.float32)
    for _ in range(3):
        y = 0.5 * (y + v / y)
    return y


def _sc_body(rows_per_w, x_hbm, lab_hbm, cen_hbm, s_hbm,
             x_v, lab_v, cen_v, out_v):
    wid = lax.axis_index("s") * _NC + lax.axis_index("c")
    base = wid * rows_per_w
    pltpu.sync_copy(cen_hbm, cen_v)
    pltpu.sync_copy(lab_hbm.at[pl.ds(base, rows_per_w)], lab_v)
    pltpu.sync_copy(x_hbm.at[pl.ds(base, rows_per_w)], x_v)

    iota = lax.iota(jnp.int32, _L)
    zero = jnp.zeros((_L,), jnp.float32)
    one = jnp.ones((_L,), jnp.float32)

    def group(g, carry):
        s_acc, n_acc = carry
        lv = lab_v[pl.ds(g * _L, _L)]
        rows = g * _L + iota
        acc = zero
        for d in range(_F):
            dvec = (8 * iota + d) & (_F - 1)
            xv = plsc.load_gather(x_v, [rows, dvec])
            cv = plsc.load_gather(cen_v, [lv, dvec])
            t = xv - cv
            acc = acc + t * t
        dist = _sqrt16(acc)
        new_s = []
        new_n = []
        for cls in range(_C):
            m = lv == cls
            new_s.append(s_acc[cls] + jnp.where(m, dist, zero))
            new_n.append(n_acc[cls] + jnp.where(m, one, zero))
        return tuple(new_s), tuple(new_n)

    init = (tuple(jnp.zeros((_L,), jnp.float32) for _ in range(_C)),) * 2
    s_acc, n_acc = lax.fori_loop(0, rows_per_w // _L, group, init)

    s_vec = zero
    n_vec = zero
    for cls in range(_C):
        s_tot = jnp.sum(s_acc[cls])
        n_tot = jnp.sum(n_acc[cls])
        sel = iota == cls
        s_vec = jnp.where(sel, jnp.full((_L,), s_tot), s_vec)
        n_vec = jnp.where(sel, jnp.full((_L,), n_tot), n_vec)
    out_v[pl.ds(0, _L)] = s_vec
    out_v[pl.ds(_L, _L)] = n_vec
    pltpu.sync_copy(out_v, s_hbm.at[wid])


def _sc_call(x, labels, centers):
    batch = x.shape[0]
    rows_per_w = batch // _NW
    mesh = plsc.VectorSubcoreMesh(core_axis_name="c", subcore_axis_name="s")
    kfn = functools.partial(_sc_body, rows_per_w)
    run = pl.kernel(
        kfn,
        mesh=mesh,
        out_type=jax.ShapeDtypeStruct((_NW, 2 * _L), jnp.float32),
        scratch_types=[
            pltpu.VMEM((rows_per_w, _F), jnp.float32),
            pltpu.VMEM((rows_per_w,), jnp.int32),
            pltpu.VMEM((_C, _F), jnp.float32),
            pltpu.VMEM((2 * _L,), jnp.float32),
        ],
        compiler_params=pltpu.CompilerParams(needs_layout_passes=False),
    )
    sn = run(x, labels.astype(jnp.int32), centers)
    return sn


def kernel(x, labels, centers):
    sn = _sc_call(x, labels, centers)
    s = jnp.sum(sn[:, :_L], axis=0)[:_C]
    n = jnp.sum(sn[:, _L:], axis=0)[:_C]
    return jnp.sum(jnp.where(n > 0, s / n, 0.0))


# speedup vs baseline: 1.4866x; 1.4866x over previous
import functools

import jax
import jax.numpy as jnp
from jax import lax
from jax.experimental import pallas as pl
from jax.experimental.pallas import tpu as pltpu
from jax.experimental.pallas import tpu_sc as plsc

_C = 10
_F = 128
_NC = 2
_NS = 16
_NW = _NC * _NS
_L = 16


def _sqrt16(v):
    bits = lax.bitcast_convert_type(v, jnp.int32)
    y = lax.bitcast_convert_type(
        (bits >> 1) + jnp.int32(0x1FBD1DF5), jnp.float32)
    for _ in range(3):
        y = 0.5 * (y + v / y)
    return y


def _sc_body(rows_per_w, x_hbm, lab_hbm, cen_hbm, s_hbm,
             x_v, lab_v, cen_v, s_v, n_v, out_v):
    wid = lax.axis_index("s") * _NC + lax.axis_index("c")
    base = wid * rows_per_w
    pltpu.sync_copy(cen_hbm, cen_v)
    pltpu.sync_copy(lab_hbm.at[pl.ds(base, rows_per_w)], lab_v)
    pltpu.sync_copy(x_hbm.at[pl.ds(base, rows_per_w)], x_v)

    iota = lax.iota(jnp.int32, _L)
    zero = jnp.zeros((_L,), jnp.float32)
    one = jnp.ones((_L,), jnp.float32)
    mall = iota >= 0

    for k in range(_L):
        s_v[k, pl.ds(0, _L)] = zero
        n_v[k, pl.ds(0, _L)] = zero

    def group(g, _):
        lv = lab_v[pl.ds(g * _L, _L)]
        rows = g * _L + iota
        acc0 = zero
        acc1 = zero
        for d in range(0, _F, 2):
            dv0 = (iota + d) & (_F - 1)
            dv1 = (iota + d + 1) & (_F - 1)
            xv0 = plsc.load_gather(x_v, [rows, dv0], mask=mall)
            cv0 = plsc.load_gather(cen_v, [lv, dv0], mask=mall)
            xv1 = plsc.load_gather(x_v, [rows, dv1], mask=mall)
            cv1 = plsc.load_gather(cen_v, [lv, dv1], mask=mall)
            t0 = xv0 - cv0
            t1 = xv1 - cv1
            acc0 = acc0 + t0 * t0
            acc1 = acc1 + t1 * t1
        dist = _sqrt16(acc0 + acc1)
        plsc.addupdate_scatter(s_v, [iota, lv], dist, mask=mall)
        plsc.addupdate_scatter(n_v, [iota, lv], one, mask=mall)
        return 0

    lax.fori_loop(0, rows_per_w // _L, group, 0)

    s_vec = zero
    n_vec = zero
    for k in range(_L):
        s_vec = s_vec + s_v[k, pl.ds(0, _L)]
        n_vec = n_vec + n_v[k, pl.ds(0, _L)]
    out_v[pl.ds(0, _L)] = s_vec
    out_v[pl.ds(_L, _L)] = n_vec
    pltpu.sync_copy(out_v, s_hbm.at[wid])


def _sc_call(x, labels, centers):
    batch = x.shape[0]
    rows_per_w = batch // _NW
    mesh = plsc.VectorSubcoreMesh(core_axis_name="c", subcore_axis_name="s")
    kfn = functools.partial(_sc_body, rows_per_w)
    run = pl.kernel(
        kfn,
        mesh=mesh,
        out_type=jax.ShapeDtypeStruct((_NW, 2 * _L), jnp.float32),
        scratch_types=[
            pltpu.VMEM((rows_per_w, _F), jnp.float32),
            pltpu.VMEM((rows_per_w,), jnp.int32),
            pltpu.VMEM((_C, _F), jnp.float32),
            pltpu.VMEM((_L, _L + 1), jnp.float32),
            pltpu.VMEM((_L, _L + 1), jnp.float32),
            pltpu.VMEM((2 * _L,), jnp.float32),
        ],
        compiler_params=pltpu.CompilerParams(needs_layout_passes=False),
    )
    sn = run(x, labels.astype(jnp.int32), centers)
    return sn


def kernel(x, labels, centers):
    sn = _sc_call(x, labels, centers)
    s = jnp.sum(sn[:, :_L], axis=0)[:_C]
    n = jnp.sum(sn[:, _L:], axis=0)[:_C]
    return jnp.sum(jnp.where(n > 0, s / n, 0.0))
